# Initial kernel scaffold; baseline (speedup 1.0000x reference)
import jax
import jax.numpy as jnp
from jax.experimental import pallas as pl


def kernel(x, edge_index, W1, b1, W2, b2):
    raise NotImplementedError("write your pallas kernel here")


# speedup vs baseline: 13.8044x; 13.8044x over previous
import functools

import jax
import jax.numpy as jnp
from jax import lax
from jax.experimental import pallas as pl
from jax.experimental.pallas import tpu as pltpu
from jax.experimental.pallas import tpu_sc as plsc

N_NODES = 10000
N_EDGES = 320000
F_IN = 128
F_HID = 128
F_OUT = 64

NC = 2
NS = 16
NW = NC * NS
PER_W = N_EDGES // NW
CHUNK = 80
N_CHUNKS = PER_W // CHUNK
SUB_ROWS = 624
ZROWS = 104
N_SLABS = SUB_ROWS // ZROWS
TAIL_OFF = NS * SUB_ROWS
TAIL = N_NODES - TAIL_OFF

_MESH = plsc.VectorSubcoreMesh(
    core_axis_name="c", subcore_axis_name="s", num_cores=NC, num_subcores=NS
)

_SC_PARAMS = pltpu.CompilerParams(use_tc_tiling_on_sc=False)


def _sc_segment_add(width):
  @functools.partial(
      pl.kernel,
      out_type=jax.ShapeDtypeStruct((NC, N_NODES, width), jnp.float32),
      mesh=_MESH,
      compiler_params=_SC_PARAMS,
      scratch_types=[
          pltpu.VMEM((CHUNK,), jnp.int32),
          pltpu.VMEM((CHUNK,), jnp.int32),
          pltpu.VMEM((ZROWS, width), jnp.float32),
          pltpu.VMEM_SHARED((N_NODES, width), jnp.float32),
          pltpu.SemaphoreType.DMA,
      ],
  )
  def k(h_hbm, src_hbm, dst_hbm, out_hbm, srcv, dstv, rows, acc, sem):
    cid = lax.axis_index("c")
    sid = lax.axis_index("s")
    wid = sid * NC + cid

    @pl.loop(0, ZROWS)
    def _(r):
      @pl.loop(0, width // 16)
      def _(c):
        rows[r, pl.ds(c * 16, 16)] = jnp.zeros((16,), jnp.float32)

    @pl.loop(0, N_SLABS)
    def _(i):
      pltpu.sync_copy(rows, acc.at[pl.ds(sid * SUB_ROWS + i * ZROWS, ZROWS)])

    @pl.when(sid == 0)
    def _():
      pltpu.sync_copy(rows.at[pl.ds(0, TAIL)], acc.at[pl.ds(TAIL_OFF, TAIL)])

    plsc.subcore_barrier()

    @pl.loop(0, N_CHUNKS)
    def _(i):
      base = wid * PER_W + i * CHUNK
      pltpu.sync_copy(src_hbm.at[pl.ds(base, CHUNK)], srcv)
      pltpu.sync_copy(dst_hbm.at[pl.ds(base, CHUNK)], dstv)
      pltpu.async_copy(h_hbm.at[srcv], rows.at[pl.ds(0, CHUNK)], sem).wait()
      pltpu.sync_copy(rows.at[pl.ds(0, CHUNK)], acc.at[dstv], add=True)

    plsc.subcore_barrier()

    pltpu.sync_copy(
        acc.at[pl.ds(sid * SUB_ROWS, SUB_ROWS)],
        out_hbm.at[cid].at[pl.ds(sid * SUB_ROWS, SUB_ROWS)],
    )

    @pl.when(sid == 0)
    def _():
      pltpu.sync_copy(
          acc.at[pl.ds(TAIL_OFF, TAIL)],
          out_hbm.at[cid].at[pl.ds(TAIL_OFF, TAIL)],
      )

  return k


_DEG_W = 16


@functools.partial(
    pl.kernel,
    out_type=jax.ShapeDtypeStruct((NC, N_NODES, _DEG_W), jnp.float32),
    mesh=_MESH,
    compiler_params=_SC_PARAMS,
    scratch_types=[
        pltpu.VMEM((CHUNK,), jnp.int32),
        pltpu.VMEM((ZROWS, _DEG_W), jnp.float32),
        pltpu.VMEM((CHUNK, _DEG_W), jnp.float32),
        pltpu.VMEM_SHARED((N_NODES, _DEG_W), jnp.float32),
        pltpu.SemaphoreType.DMA,
    ],
)
def _sc_degree(dst_hbm, out_hbm, dstv, zbuf, ones, acc, sem):
  cid = lax.axis_index("c")
  sid = lax.axis_index("s")
  wid = sid * NC + cid

  @pl.loop(0, ZROWS)
  def _(r):
    zbuf[r, pl.ds(0, 16)] = jnp.zeros((16,), jnp.float32)

  @pl.loop(0, CHUNK)
  def _(r):
    ones[r, pl.ds(0, 16)] = jnp.ones((16,), jnp.float32)

  @pl.loop(0, N_SLABS)
  def _(i):
    pltpu.sync_copy(zbuf, acc.at[pl.ds(sid * SUB_ROWS + i * ZROWS, ZROWS)])

  @pl.when(sid == 0)
  def _():
    pltpu.sync_copy(zbuf.at[pl.ds(0, TAIL)], acc.at[pl.ds(TAIL_OFF, TAIL)])

  plsc.subcore_barrier()

  @pl.loop(0, N_CHUNKS)
  def _(i):
    base = wid * PER_W + i * CHUNK
    pltpu.sync_copy(dst_hbm.at[pl.ds(base, CHUNK)], dstv)
    pltpu.sync_copy(ones, acc.at[dstv], add=True)

  plsc.subcore_barrier()

  pltpu.sync_copy(
      acc.at[pl.ds(sid * SUB_ROWS, SUB_ROWS)],
      out_hbm.at[cid].at[pl.ds(sid * SUB_ROWS, SUB_ROWS)],
  )

  @pl.when(sid == 0)
  def _():
    pltpu.sync_copy(
        acc.at[pl.ds(TAIL_OFF, TAIL)],
        out_hbm.at[cid].at[pl.ds(TAIL_OFF, TAIL)],
    )


_BLK = 1000
_GRID = N_NODES // _BLK


def _tc_matmul(x, w):
  k, m = w.shape

  def body(x_ref, w_ref, o_ref):
    o_ref[...] = jnp.dot(
        x_ref[...], w_ref[...], preferred_element_type=jnp.float32
    )

  return pl.pallas_call(
      body,
      grid=(_GRID,),
      in_specs=[
          pl.BlockSpec((_BLK, k), lambda i: (i, 0)),
          pl.BlockSpec((k, m), lambda i: (0, 0)),
      ],
      out_specs=pl.BlockSpec((_BLK, m), lambda i: (i, 0)),
      out_shape=jax.ShapeDtypeStruct((N_NODES, m), jnp.float32),
  )(x, w)


def _dinv_of(degp_ref):
  d = degp_ref[0, :, 0] + degp_ref[1, :, 0] + 1.0
  return lax.rsqrt(d)[:, None]


def _tc_scale(degp, h):
  m = h.shape[1]

  def body(degp_ref, h_ref, o_ref):
    o_ref[...] = h_ref[...] * _dinv_of(degp_ref)

  return pl.pallas_call(
      body,
      grid=(_GRID,),
      in_specs=[
          pl.BlockSpec((NC, _BLK, _DEG_W), lambda i: (0, i, 0)),
          pl.BlockSpec((_BLK, m), lambda i: (i, 0)),
      ],
      out_specs=pl.BlockSpec((_BLK, m), lambda i: (i, 0)),
      out_shape=jax.ShapeDtypeStruct((N_NODES, m), jnp.float32),
  )(degp, h)


def _tc_layer1_finish(aggp, h1, degp, b1, w2):
  def body(aggp_ref, h1_ref, degp_ref, b1_ref, w2_ref, h2_ref, h2p_ref):
    dinv = _dinv_of(degp_ref)
    y = dinv * (aggp_ref[0] + aggp_ref[1]) + (dinv * dinv) * h1_ref[...]
    y = jnp.maximum(y + b1_ref[...][None, :], 0.0)
    h2 = jnp.dot(y, w2_ref[...], preferred_element_type=jnp.float32)
    h2_ref[...] = h2
    h2p_ref[...] = h2 * dinv

  return pl.pallas_call(
      body,
      grid=(_GRID,),
      in_specs=[
          pl.BlockSpec((NC, _BLK, F_HID), lambda i: (0, i, 0)),
          pl.BlockSpec((_BLK, F_HID), lambda i: (i, 0)),
          pl.BlockSpec((NC, _BLK, _DEG_W), lambda i: (0, i, 0)),
          pl.BlockSpec((F_HID,), lambda i: (0,)),
          pl.BlockSpec((F_HID, F_OUT), lambda i: (0, 0)),
      ],
      out_specs=[
          pl.BlockSpec((_BLK, F_OUT), lambda i: (i, 0)),
          pl.BlockSpec((_BLK, F_OUT), lambda i: (i, 0)),
      ],
      out_shape=[
          jax.ShapeDtypeStruct((N_NODES, F_OUT), jnp.float32),
          jax.ShapeDtypeStruct((N_NODES, F_OUT), jnp.float32),
      ],
  )(aggp, h1, degp, b1, w2)


def _tc_layer2_finish(aggp, h2, degp, b2):
  def body(aggp_ref, h2_ref, degp_ref, b2_ref, o_ref):
    dinv = _dinv_of(degp_ref)
    z = dinv * (aggp_ref[0] + aggp_ref[1]) + (dinv * dinv) * h2_ref[...]
    z = z + b2_ref[...][None, :]
    m = jnp.max(z, axis=1, keepdims=True)
    e = z - m
    o_ref[...] = e - jnp.log(jnp.sum(jnp.exp(e), axis=1, keepdims=True))

  return pl.pallas_call(
      body,
      grid=(_GRID,),
      in_specs=[
          pl.BlockSpec((NC, _BLK, F_OUT), lambda i: (0, i, 0)),
          pl.BlockSpec((_BLK, F_OUT), lambda i: (i, 0)),
          pl.BlockSpec((NC, _BLK, _DEG_W), lambda i: (0, i, 0)),
          pl.BlockSpec((F_OUT,), lambda i: (0,)),
      ],
      out_specs=pl.BlockSpec((_BLK, F_OUT), lambda i: (i, 0)),
      out_shape=jax.ShapeDtypeStruct((N_NODES, F_OUT), jnp.float32),
  )(aggp, h2, degp, b2)


_agg128 = _sc_segment_add(F_HID)
_agg64 = _sc_segment_add(F_OUT)


def kernel(x, edge_index, W1, b1, W2, b2):
  src = edge_index[0].astype(jnp.int32)
  dst = edge_index[1].astype(jnp.int32)

  degp = _sc_degree(dst)
  h1 = _tc_matmul(x, W1)
  h1p = _tc_scale(degp, h1)
  agg1 = _agg128(h1p, src, dst)
  h2, h2p = _tc_layer1_finish(agg1, h1, degp, b1, W2)
  agg2 = _agg64(h2p, src, dst)
  return _tc_layer2_finish(agg2, h2, degp, b2)


# speedup vs baseline: 30.0768x; 2.1788x over previous
import functools

import jax
import jax.numpy as jnp
from jax import lax
from jax.experimental import pallas as pl
from jax.experimental.pallas import tpu as pltpu
from jax.experimental.pallas import tpu_sc as plsc

N_NODES = 10000
N_EDGES = 320000
F_IN = 128
F_HID = 128
F_OUT = 64

NC = 2
NS = 16
NW = NC * NS
PER_W = N_EDGES // NW
CHUNK = 100
N_CHUNKS = PER_W // CHUNK
NBUF_BY_WIDTH = {128: 2, 64: 5}
NBUF_DEG = 5
SUB_ROWS = 624
ZROWS = 48
N_SLABS = SUB_ROWS // ZROWS
TAIL_OFF = NS * SUB_ROWS
TAIL = N_NODES - TAIL_OFF

_MESH = plsc.VectorSubcoreMesh(
    core_axis_name="c", subcore_axis_name="s", num_cores=NC, num_subcores=NS
)

_SC_PARAMS = pltpu.CompilerParams(use_tc_tiling_on_sc=False)


def _sc_segment_add(width):
  nbuf = NBUF_BY_WIDTH[width]
  n_bodies = N_CHUNKS // nbuf

  @functools.partial(
      pl.kernel,
      out_type=jax.ShapeDtypeStruct((NC, N_NODES, width), jnp.float32),
      mesh=_MESH,
      compiler_params=_SC_PARAMS,
      scratch_types=[
          pltpu.VMEM((N_CHUNKS, CHUNK), jnp.int32),
          pltpu.VMEM((N_CHUNKS, CHUNK), jnp.int32),
      ]
      + [pltpu.VMEM((CHUNK, width), jnp.float32) for _ in range(nbuf)]
      + [
          pltpu.VMEM_SHARED((N_NODES, width), jnp.float32),
          pltpu.SemaphoreType.DMA,
          pltpu.SemaphoreType.DMA((nbuf,)),
          pltpu.SemaphoreType.DMA,
      ],
  )
  def k(h_hbm, src_hbm, dst_hbm, out_hbm, *rest):
    srcv, dstv = rest[0], rest[1]
    bufs = list(rest[2:2 + nbuf])
    acc, isem, gsem, ssem = rest[2 + nbuf:]
    b0 = bufs[0]
    cid = lax.axis_index("c")
    sid = lax.axis_index("s")
    wid = sid * NC + cid

    di_s = pltpu.async_copy(src_hbm.at[wid], srcv, isem)
    di_d = pltpu.async_copy(dst_hbm.at[wid], dstv, isem)

    @pl.loop(0, ZROWS)
    def _(r):
      @pl.loop(0, width // 16)
      def _(c):
        b0[r, pl.ds(c * 16, 16)] = jnp.zeros((16,), jnp.float32)

    @pl.loop(0, N_SLABS)
    def _(i):
      pltpu.sync_copy(
          b0.at[pl.ds(0, ZROWS)],
          acc.at[pl.ds(sid * SUB_ROWS + i * ZROWS, ZROWS)],
      )

    @pl.when(sid == 0)
    def _():
      pltpu.sync_copy(b0.at[pl.ds(0, TAIL)], acc.at[pl.ds(TAIL_OFF, TAIL)])

    di_s.wait()
    di_d.wait()
    plsc.subcore_barrier()

    @pl.loop(0, n_bodies)
    def _(j):
      c0 = j * nbuf
      gds = [
          pltpu.async_copy(h_hbm.at[srcv.at[c0 + b]], bufs[b], gsem.at[b])
          for b in range(nbuf)
      ]
      sds = []
      for b in range(nbuf):
        gds[b].wait()
        sds.append(
            pltpu.async_copy(bufs[b], acc.at[dstv.at[c0 + b]], ssem, add=True)
        )
      for d in sds:
        d.wait()

    plsc.subcore_barrier()

    pltpu.sync_copy(
        acc.at[pl.ds(sid * SUB_ROWS, SUB_ROWS)],
        out_hbm.at[cid].at[pl.ds(sid * SUB_ROWS, SUB_ROWS)],
    )

    @pl.when(sid == 0)
    def _():
      pltpu.sync_copy(
          acc.at[pl.ds(TAIL_OFF, TAIL)],
          out_hbm.at[cid].at[pl.ds(TAIL_OFF, TAIL)],
      )

  return k


_DEG_W = 16


@functools.partial(
    pl.kernel,
    out_type=jax.ShapeDtypeStruct((NC, N_NODES, _DEG_W), jnp.float32),
    mesh=_MESH,
    compiler_params=_SC_PARAMS,
    scratch_types=[
        pltpu.VMEM((N_CHUNKS, CHUNK), jnp.int32),
        pltpu.VMEM((CHUNK, _DEG_W), jnp.float32),
        pltpu.VMEM_SHARED((N_NODES, _DEG_W), jnp.float32),
        pltpu.SemaphoreType.DMA,
        pltpu.SemaphoreType.DMA,
    ],
)
def _sc_degree(dst_hbm, out_hbm, dstv, ones, acc, isem, ssem):
  cid = lax.axis_index("c")
  sid = lax.axis_index("s")
  wid = sid * NC + cid

  di = pltpu.async_copy(dst_hbm.at[wid], dstv, isem)

  @pl.loop(0, CHUNK)
  def _(r):
    ones[r, pl.ds(0, 16)] = jnp.zeros((16,), jnp.float32)

  @pl.loop(0, N_SLABS)
  def _(i):
    pltpu.sync_copy(
        ones.at[pl.ds(0, ZROWS)],
        acc.at[pl.ds(sid * SUB_ROWS + i * ZROWS, ZROWS)],
    )

  @pl.when(sid == 0)
  def _():
    pltpu.sync_copy(ones.at[pl.ds(0, TAIL)], acc.at[pl.ds(TAIL_OFF, TAIL)])

  @pl.loop(0, CHUNK)
  def _(r):
    ones[r, pl.ds(0, 16)] = jnp.ones((16,), jnp.float32)

  di.wait()
  plsc.subcore_barrier()

  @pl.loop(0, N_CHUNKS // NBUF_DEG)
  def _(j):
    sds = [
        pltpu.async_copy(ones, acc.at[dstv.at[j * NBUF_DEG + b]], ssem, add=True)
        for b in range(NBUF_DEG)
    ]
    for d in sds:
      d.wait()

  plsc.subcore_barrier()

  pltpu.sync_copy(
      acc.at[pl.ds(sid * SUB_ROWS, SUB_ROWS)],
      out_hbm.at[cid].at[pl.ds(sid * SUB_ROWS, SUB_ROWS)],
  )

  @pl.when(sid == 0)
  def _():
    pltpu.sync_copy(
        acc.at[pl.ds(TAIL_OFF, TAIL)],
        out_hbm.at[cid].at[pl.ds(TAIL_OFF, TAIL)],
    )


_BLK = 1000
_GRID = N_NODES // _BLK


def _tc_matmul(x, w):
  k, m = w.shape

  def body(x_ref, w_ref, o_ref):
    o_ref[...] = jnp.dot(
        x_ref[...], w_ref[...], preferred_element_type=jnp.float32
    )

  return pl.pallas_call(
      body,
      grid=(_GRID,),
      in_specs=[
          pl.BlockSpec((_BLK, k), lambda i: (i, 0)),
          pl.BlockSpec((k, m), lambda i: (0, 0)),
      ],
      out_specs=pl.BlockSpec((_BLK, m), lambda i: (i, 0)),
      out_shape=jax.ShapeDtypeStruct((N_NODES, m), jnp.float32),
  )(x, w)


def _dinv_of(degp_ref):
  d = degp_ref[0, :, 0] + degp_ref[1, :, 0] + 1.0
  return lax.rsqrt(d)[:, None]


def _tc_scale(degp, h):
  m = h.shape[1]

  def body(degp_ref, h_ref, o_ref):
    o_ref[...] = h_ref[...] * _dinv_of(degp_ref)

  return pl.pallas_call(
      body,
      grid=(_GRID,),
      in_specs=[
          pl.BlockSpec((NC, _BLK, _DEG_W), lambda i: (0, i, 0)),
          pl.BlockSpec((_BLK, m), lambda i: (i, 0)),
      ],
      out_specs=pl.BlockSpec((_BLK, m), lambda i: (i, 0)),
      out_shape=jax.ShapeDtypeStruct((N_NODES, m), jnp.float32),
  )(degp, h)


def _tc_layer1_finish(aggp, h1, degp, b1, w2):
  def body(aggp_ref, h1_ref, degp_ref, b1_ref, w2_ref, h2_ref, h2p_ref):
    dinv = _dinv_of(degp_ref)
    y = dinv * (aggp_ref[0] + aggp_ref[1]) + (dinv * dinv) * h1_ref[...]
    y = jnp.maximum(y + b1_ref[...][None, :], 0.0)
    h2 = jnp.dot(y, w2_ref[...], preferred_element_type=jnp.float32)
    h2_ref[...] = h2
    h2p_ref[...] = h2 * dinv

  return pl.pallas_call(
      body,
      grid=(_GRID,),
      in_specs=[
          pl.BlockSpec((NC, _BLK, F_HID), lambda i: (0, i, 0)),
          pl.BlockSpec((_BLK, F_HID), lambda i: (i, 0)),
          pl.BlockSpec((NC, _BLK, _DEG_W), lambda i: (0, i, 0)),
          pl.BlockSpec((F_HID,), lambda i: (0,)),
          pl.BlockSpec((F_HID, F_OUT), lambda i: (0, 0)),
      ],
      out_specs=[
          pl.BlockSpec((_BLK, F_OUT), lambda i: (i, 0)),
          pl.BlockSpec((_BLK, F_OUT), lambda i: (i, 0)),
      ],
      out_shape=[
          jax.ShapeDtypeStruct((N_NODES, F_OUT), jnp.float32),
          jax.ShapeDtypeStruct((N_NODES, F_OUT), jnp.float32),
      ],
  )(aggp, h1, degp, b1, w2)


def _tc_layer2_finish(aggp, h2, degp, b2):
  def body(aggp_ref, h2_ref, degp_ref, b2_ref, o_ref):
    dinv = _dinv_of(degp_ref)
    z = dinv * (aggp_ref[0] + aggp_ref[1]) + (dinv * dinv) * h2_ref[...]
    z = z + b2_ref[...][None, :]
    m = jnp.max(z, axis=1, keepdims=True)
    e = z - m
    o_ref[...] = e - jnp.log(jnp.sum(jnp.exp(e), axis=1, keepdims=True))

  return pl.pallas_call(
      body,
      grid=(_GRID,),
      in_specs=[
          pl.BlockSpec((NC, _BLK, F_OUT), lambda i: (0, i, 0)),
          pl.BlockSpec((_BLK, F_OUT), lambda i: (i, 0)),
          pl.BlockSpec((NC, _BLK, _DEG_W), lambda i: (0, i, 0)),
          pl.BlockSpec((F_OUT,), lambda i: (0,)),
      ],
      out_specs=pl.BlockSpec((_BLK, F_OUT), lambda i: (i, 0)),
      out_shape=jax.ShapeDtypeStruct((N_NODES, F_OUT), jnp.float32),
  )(aggp, h2, degp, b2)


_agg128 = _sc_segment_add(F_HID)
_agg64 = _sc_segment_add(F_OUT)


def kernel(x, edge_index, W1, b1, W2, b2):
  src = edge_index[0].astype(jnp.int32).reshape(NW, N_CHUNKS, CHUNK)
  dst = edge_index[1].astype(jnp.int32).reshape(NW, N_CHUNKS, CHUNK)

  degp = _sc_degree(dst)
  h1 = _tc_matmul(x, W1)
  h1p = _tc_scale(degp, h1)
  agg1 = _agg128(h1p, src, dst)
  h2, h2p = _tc_layer1_finish(agg1, h1, degp, b1, W2)
  agg2 = _agg64(h2p, src, dst)
  return _tc_layer2_finish(agg2, h2, degp, b2)


# speedup vs baseline: 30.4336x; 1.0119x over previous
import functools

import jax
import jax.numpy as jnp
from jax import lax
from jax.experimental import pallas as pl
from jax.experimental.pallas import tpu as pltpu
from jax.experimental.pallas import tpu_sc as plsc

N_NODES = 10000
N_EDGES = 320000
F_IN = 128
F_HID = 128
F_OUT = 64

NC = 2
NS = 16
NW = NC * NS
PER_W = N_EDGES // NW
CHUNK = 100
N_CHUNKS = PER_W // CHUNK
NBUF_BY_WIDTH = {128: 2, 64: 4}
NBUF_DEG = 5
SUB_ROWS = 624
ZROWS = 48
N_SLABS = SUB_ROWS // ZROWS
TAIL_OFF = NS * SUB_ROWS
TAIL = N_NODES - TAIL_OFF

_MESH = plsc.VectorSubcoreMesh(
    core_axis_name="c", subcore_axis_name="s", num_cores=NC, num_subcores=NS
)

_SC_PARAMS = pltpu.CompilerParams(use_tc_tiling_on_sc=False)


def _sc_segment_add(width, spmem_table=False):
  nbuf = NBUF_BY_WIDTH[width]
  n_bodies = N_CHUNKS // nbuf

  @functools.partial(
      pl.kernel,
      out_type=jax.ShapeDtypeStruct((NC, N_NODES, width), jnp.float32),
      mesh=_MESH,
      compiler_params=_SC_PARAMS,
      scratch_types=[
          pltpu.VMEM((N_CHUNKS, CHUNK), jnp.int32),
          pltpu.VMEM((N_CHUNKS, CHUNK), jnp.int32),
      ]
      + [pltpu.VMEM((CHUNK, width), jnp.float32) for _ in range(nbuf)]
      + ([pltpu.VMEM_SHARED((N_NODES, width), jnp.float32)]
         if spmem_table else [])
      + [
          pltpu.VMEM_SHARED((N_NODES, width), jnp.float32),
          pltpu.SemaphoreType.DMA,
          pltpu.SemaphoreType.DMA((nbuf,)),
          pltpu.SemaphoreType.DMA,
      ],
  )
  def k(h_hbm, src_hbm, dst_hbm, out_hbm, *rest):
    srcv, dstv = rest[0], rest[1]
    bufs = list(rest[2:2 + nbuf])
    rest = rest[2 + nbuf:]
    if spmem_table:
      tbl, acc, isem, gsem, ssem = rest
    else:
      acc, isem, gsem, ssem = rest
      tbl = None
    b0 = bufs[0]
    cid = lax.axis_index("c")
    sid = lax.axis_index("s")
    wid = sid * NC + cid

    di_s = pltpu.async_copy(src_hbm.at[wid], srcv, isem)
    di_d = pltpu.async_copy(dst_hbm.at[wid], dstv, isem)

    if spmem_table:
      pltpu.sync_copy(
          h_hbm.at[pl.ds(sid * SUB_ROWS, SUB_ROWS)],
          tbl.at[pl.ds(sid * SUB_ROWS, SUB_ROWS)],
      )
      @pl.when(sid == 0)
      def _():
        pltpu.sync_copy(
            h_hbm.at[pl.ds(TAIL_OFF, TAIL)], tbl.at[pl.ds(TAIL_OFF, TAIL)]
        )
    gather_src = tbl if spmem_table else h_hbm

    @pl.loop(0, ZROWS)
    def _(r):
      @pl.loop(0, width // 16)
      def _(c):
        b0[r, pl.ds(c * 16, 16)] = jnp.zeros((16,), jnp.float32)

    @pl.loop(0, N_SLABS)
    def _(i):
      pltpu.sync_copy(
          b0.at[pl.ds(0, ZROWS)],
          acc.at[pl.ds(sid * SUB_ROWS + i * ZROWS, ZROWS)],
      )

    @pl.when(sid == 0)
    def _():
      pltpu.sync_copy(b0.at[pl.ds(0, TAIL)], acc.at[pl.ds(TAIL_OFF, TAIL)])

    di_s.wait()
    di_d.wait()
    plsc.subcore_barrier()

    @pl.loop(0, n_bodies)
    def _(j):
      c0 = j * nbuf
      gds = [
          pltpu.async_copy(gather_src.at[srcv.at[c0 + b]], bufs[b], gsem.at[b])
          for b in range(nbuf)
      ]
      sds = []
      for b in range(nbuf):
        gds[b].wait()
        sds.append(
            pltpu.async_copy(bufs[b], acc.at[dstv.at[c0 + b]], ssem, add=True)
        )
      for d in sds:
        d.wait()

    plsc.subcore_barrier()

    pltpu.sync_copy(
        acc.at[pl.ds(sid * SUB_ROWS, SUB_ROWS)],
        out_hbm.at[cid].at[pl.ds(sid * SUB_ROWS, SUB_ROWS)],
    )

    @pl.when(sid == 0)
    def _():
      pltpu.sync_copy(
          acc.at[pl.ds(TAIL_OFF, TAIL)],
          out_hbm.at[cid].at[pl.ds(TAIL_OFF, TAIL)],
      )

  return k


_DEG_W = 16


@functools.partial(
    pl.kernel,
    out_type=jax.ShapeDtypeStruct((NC, N_NODES, _DEG_W), jnp.float32),
    mesh=_MESH,
    compiler_params=_SC_PARAMS,
    scratch_types=[
        pltpu.VMEM((N_CHUNKS, CHUNK), jnp.int32),
        pltpu.VMEM((CHUNK, _DEG_W), jnp.float32),
        pltpu.VMEM_SHARED((N_NODES, _DEG_W), jnp.float32),
        pltpu.SemaphoreType.DMA,
        pltpu.SemaphoreType.DMA,
    ],
)
def _sc_degree(dst_hbm, out_hbm, dstv, ones, acc, isem, ssem):
  cid = lax.axis_index("c")
  sid = lax.axis_index("s")
  wid = sid * NC + cid

  di = pltpu.async_copy(dst_hbm.at[wid], dstv, isem)

  @pl.loop(0, CHUNK)
  def _(r):
    ones[r, pl.ds(0, 16)] = jnp.zeros((16,), jnp.float32)

  @pl.loop(0, N_SLABS)
  def _(i):
    pltpu.sync_copy(
        ones.at[pl.ds(0, ZROWS)],
        acc.at[pl.ds(sid * SUB_ROWS + i * ZROWS, ZROWS)],
    )

  @pl.when(sid == 0)
  def _():
    pltpu.sync_copy(ones.at[pl.ds(0, TAIL)], acc.at[pl.ds(TAIL_OFF, TAIL)])

  @pl.loop(0, CHUNK)
  def _(r):
    ones[r, pl.ds(0, 16)] = jnp.ones((16,), jnp.float32)

  di.wait()
  plsc.subcore_barrier()

  @pl.loop(0, N_CHUNKS // NBUF_DEG)
  def _(j):
    sds = [
        pltpu.async_copy(ones, acc.at[dstv.at[j * NBUF_DEG + b]], ssem, add=True)
        for b in range(NBUF_DEG)
    ]
    for d in sds:
      d.wait()

  plsc.subcore_barrier()

  pltpu.sync_copy(
      acc.at[pl.ds(sid * SUB_ROWS, SUB_ROWS)],
      out_hbm.at[cid].at[pl.ds(sid * SUB_ROWS, SUB_ROWS)],
  )

  @pl.when(sid == 0)
  def _():
    pltpu.sync_copy(
        acc.at[pl.ds(TAIL_OFF, TAIL)],
        out_hbm.at[cid].at[pl.ds(TAIL_OFF, TAIL)],
    )


_BLK = 1000
_GRID = N_NODES // _BLK


def _tc_matmul(x, w):
  k, m = w.shape

  def body(x_ref, w_ref, o_ref):
    o_ref[...] = jnp.dot(
        x_ref[...], w_ref[...], preferred_element_type=jnp.float32
    )

  return pl.pallas_call(
      body,
      grid=(_GRID,),
      in_specs=[
          pl.BlockSpec((_BLK, k), lambda i: (i, 0)),
          pl.BlockSpec((k, m), lambda i: (0, 0)),
      ],
      out_specs=pl.BlockSpec((_BLK, m), lambda i: (i, 0)),
      out_shape=jax.ShapeDtypeStruct((N_NODES, m), jnp.float32),
  )(x, w)


def _dinv_of(degp_ref):
  d = degp_ref[0, :, 0] + degp_ref[1, :, 0] + 1.0
  return lax.rsqrt(d)[:, None]


def _tc_scale(degp, h):
  m = h.shape[1]

  def body(degp_ref, h_ref, o_ref):
    o_ref[...] = h_ref[...] * _dinv_of(degp_ref)

  return pl.pallas_call(
      body,
      grid=(_GRID,),
      in_specs=[
          pl.BlockSpec((NC, _BLK, _DEG_W), lambda i: (0, i, 0)),
          pl.BlockSpec((_BLK, m), lambda i: (i, 0)),
      ],
      out_specs=pl.BlockSpec((_BLK, m), lambda i: (i, 0)),
      out_shape=jax.ShapeDtypeStruct((N_NODES, m), jnp.float32),
  )(degp, h)


def _tc_layer1_finish(aggp, h1, degp, b1, w2):
  def body(aggp_ref, h1_ref, degp_ref, b1_ref, w2_ref, h2_ref, h2p_ref):
    dinv = _dinv_of(degp_ref)
    y = dinv * (aggp_ref[0] + aggp_ref[1]) + (dinv * dinv) * h1_ref[...]
    y = jnp.maximum(y + b1_ref[...][None, :], 0.0)
    h2 = jnp.dot(y, w2_ref[...], preferred_element_type=jnp.float32)
    h2_ref[...] = h2
    h2p_ref[...] = h2 * dinv

  return pl.pallas_call(
      body,
      grid=(_GRID,),
      in_specs=[
          pl.BlockSpec((NC, _BLK, F_HID), lambda i: (0, i, 0)),
          pl.BlockSpec((_BLK, F_HID), lambda i: (i, 0)),
          pl.BlockSpec((NC, _BLK, _DEG_W), lambda i: (0, i, 0)),
          pl.BlockSpec((F_HID,), lambda i: (0,)),
          pl.BlockSpec((F_HID, F_OUT), lambda i: (0, 0)),
      ],
      out_specs=[
          pl.BlockSpec((_BLK, F_OUT), lambda i: (i, 0)),
          pl.BlockSpec((_BLK, F_OUT), lambda i: (i, 0)),
      ],
      out_shape=[
          jax.ShapeDtypeStruct((N_NODES, F_OUT), jnp.float32),
          jax.ShapeDtypeStruct((N_NODES, F_OUT), jnp.float32),
      ],
  )(aggp, h1, degp, b1, w2)


def _tc_layer2_finish(aggp, h2, degp, b2):
  def body(aggp_ref, h2_ref, degp_ref, b2_ref, o_ref):
    dinv = _dinv_of(degp_ref)
    z = dinv * (aggp_ref[0] + aggp_ref[1]) + (dinv * dinv) * h2_ref[...]
    z = z + b2_ref[...][None, :]
    m = jnp.max(z, axis=1, keepdims=True)
    e = z - m
    o_ref[...] = e - jnp.log(jnp.sum(jnp.exp(e), axis=1, keepdims=True))

  return pl.pallas_call(
      body,
      grid=(_GRID,),
      in_specs=[
          pl.BlockSpec((NC, _BLK, F_OUT), lambda i: (0, i, 0)),
          pl.BlockSpec((_BLK, F_OUT), lambda i: (i, 0)),
          pl.BlockSpec((NC, _BLK, _DEG_W), lambda i: (0, i, 0)),
          pl.BlockSpec((F_OUT,), lambda i: (0,)),
      ],
      out_specs=pl.BlockSpec((_BLK, F_OUT), lambda i: (i, 0)),
      out_shape=jax.ShapeDtypeStruct((N_NODES, F_OUT), jnp.float32),
  )(aggp, h2, degp, b2)


_agg128 = _sc_segment_add(F_HID)
_agg64 = _sc_segment_add(F_OUT, spmem_table=True)


def kernel(x, edge_index, W1, b1, W2, b2):
  src = edge_index[0].astype(jnp.int32).reshape(NW, N_CHUNKS, CHUNK)
  dst = edge_index[1].astype(jnp.int32).reshape(NW, N_CHUNKS, CHUNK)

  degp = _sc_degree(dst)
  h1 = _tc_matmul(x, W1)
  h1p = _tc_scale(degp, h1)
  agg1 = _agg128(h1p, src, dst)
  h2, h2p = _tc_layer1_finish(agg1, h1, degp, b1, W2)
  agg2 = _agg64(h2p, src, dst)
  return _tc_layer2_finish(agg2, h2, degp, b2)
